# Initial kernel scaffold; baseline (speedup 1.0000x reference)
import jax
import jax.numpy as jnp
from jax.experimental import pallas as pl


def kernel(coarse_input, coarse_pred, fine_input, fine_pred, laplace_idx_list):
    raise NotImplementedError("write your pallas kernel here")


# speedup vs baseline: 25.0419x; 25.0419x over previous
import jax
import jax.numpy as jnp
from jax import lax
from jax.experimental import pallas as pl
from jax.experimental.pallas import tpu as pltpu
from jax.experimental.pallas import tpu_sc as plsc

N = 50000
KNB = 8
NC = 2
NS = 16
NW = NC * NS
BPW = 1568
NP = NW * BPW
VPW = BPW // 16


def _diff_body(b_ref, a_ref, *o_refs):
    for g in range(2):
        for c in range(3):
            o_refs[g * 3 + c][...] = b_ref[g, c, :] - a_ref[g, c, :]


def _sc_body(d00, d01, d02, d10, d11, d12, idx_hbm, out_hbm,
             table, idxv, outv):
    d_refs = ((d00, d01, d02), (d10, d11, d12))
    wid = lax.axis_index("c") * NS + lax.axis_index("s")
    base = wid * BPW
    lane = lax.iota(jnp.int32, 16)
    lossvec = jnp.zeros((16,), jnp.float32)
    for g in range(2):
        pltpu.sync_copy(
            idx_hbm.at[pl.ds((g * NP + base) * (KNB + 2), BPW * (KNB + 2))],
            idxv)
        for c in range(3):
            pltpu.sync_copy(d_refs[g][c], table)

            def body(v, lv):
                rows = v * 16 + lane
                nid = base + rows
                flat = rows * (KNB + 2)
                degv = plsc.load_gather(idxv, [flat + (KNB + 1)])
                inv = 1.0 / degv.astype(jnp.float32)
                acc = jnp.zeros((16,), jnp.float32)
                for k in range(KNB):
                    nb = plsc.load_gather(idxv, [flat + k])
                    acc = acc + plsc.load_gather(table, [nb])
                own = table[pl.ds(base + v * 16, 16)]
                r = own - acc * inv
                r = jnp.where(nid < N, r, 0.0)
                return lv + r * r

            lossvec = lax.fori_loop(0, VPW, body, lossvec)
    outv[...] = lossvec
    pltpu.sync_copy(outv, out_hbm.at[pl.ds(wid * 16, 16)])


def kernel(coarse_input, coarse_pred, fine_input, fine_pred, laplace_idx_list):
    pad = NP - N
    before = jnp.stack([coarse_input.T, fine_input.T])
    after = jnp.stack([coarse_pred.T, fine_pred.T])
    before = jnp.pad(before, ((0, 0), (0, 0), (0, pad)))
    after = jnp.pad(after, ((0, 0), (0, 0), (0, pad)))
    idx = jnp.pad(laplace_idx_list, ((0, 0), (0, pad), (0, 0)),
                  constant_values=1).reshape(2 * NP * (KNB + 2))

    plane = jax.ShapeDtypeStruct((NP,), jnp.float32)
    d_planes = pl.pallas_call(
        _diff_body,
        out_shape=[plane] * 6,
    )(before, after)

    mesh = plsc.VectorSubcoreMesh(core_axis_name="c", subcore_axis_name="s")
    part = pl.kernel(
        _sc_body,
        mesh=mesh,
        compiler_params=pltpu.CompilerParams(needs_layout_passes=False),
        out_type=jax.ShapeDtypeStruct((NW * 16,), jnp.float32),
        scratch_types=[
            pltpu.VMEM((NP,), jnp.float32),
            pltpu.VMEM((BPW * (KNB + 2),), jnp.int32),
            pltpu.VMEM((16,), jnp.float32),
        ],
    )(*d_planes, idx)
    return jnp.sum(part) * jnp.float32(0.5 / N)


# speedup vs baseline: 72.5627x; 2.8976x over previous
import jax
import jax.numpy as jnp
from jax import lax
from jax.experimental import pallas as pl
from jax.experimental.pallas import tpu as pltpu
from jax.experimental.pallas import tpu_sc as plsc

N = 50000
KNB = 8
NC = 2
NS = 16
NW = NC * NS
BPW = 1568
VPW = BPW // 16


def _diff_body(ci, cp, fi, fp, *o_refs):
    for c in range(3):
        o_refs[c][...] = ci[c, :] - cp[c, :]
        o_refs[3 + c][...] = fi[c, :] - fp[c, :]


def _copy_idx_columns(idxF, idxv, g, cbase):
    for slot, k in enumerate(list(range(KNB)) + [KNB + 1]):
        pltpu.sync_copy(
            idxF.at[pl.ds((k * 2 + g) * N + cbase, BPW)],
            idxv.at[pl.ds(slot * BPW, BPW)])


def _sc_body(d00, d01, d02, d10, d11, d12, idxF, out_hbm,
             table0, table1, idxv, invv, outv, sem0, sem1):
    d_planes = (d00, d01, d02, d10, d11, d12)
    bufs = (table0, table1)
    sems = (sem0, sem1)
    wid = lax.axis_index("c") * NS + lax.axis_index("s")
    base = wid * BPW
    cbase = jnp.minimum(base, N - BPW)
    doff = base - cbase
    lane = lax.iota(jnp.int32, 16)
    lossvec = jnp.zeros((16,), jnp.float32)

    pending = pltpu.async_copy(d_planes[0], bufs[0], sems[0])
    _copy_idx_columns(idxF, idxv, 0, cbase)
    for i in range(6):
        g, c = divmod(i, 3)
        nxt = None
        if i + 1 < 6:
            nxt = pltpu.async_copy(
                d_planes[i + 1], bufs[(i + 1) % 2], sems[(i + 1) % 2])
        pending.wait()
        tbl = bufs[i % 2]

        def body(v, lv, tbl=tbl, first=(c == 0)):
            o = v * 16
            if first:
                deg = idxv[pl.ds(KNB * BPW + o, 16)]
                inv = 1.0 / deg.astype(jnp.float32)
                invv[pl.ds(o, 16)] = inv
            else:
                inv = invv[pl.ds(o, 16)]
            acc = jnp.zeros((16,), jnp.float32)
            for k in range(KNB):
                nb = idxv[pl.ds(k * BPW + o, 16)]
                acc = acc + plsc.load_gather(tbl, [nb])
            own = tbl[pl.ds(cbase + o, 16)]
            r = own - acc * inv
            r = jnp.where(o + lane >= doff, r, 0.0)
            return lv + r * r

        lossvec = lax.fori_loop(0, VPW, body, lossvec)
        if i == 2:
            _copy_idx_columns(idxF, idxv, 1, cbase)
        pending = nxt
    outv[...] = lossvec
    pltpu.sync_copy(outv, out_hbm.at[pl.ds(wid * 16, 16)])


def kernel(coarse_input, coarse_pred, fine_input, fine_pred, laplace_idx_list):
    plane = jax.ShapeDtypeStruct((N,), jnp.float32)
    d_planes = pl.pallas_call(
        _diff_body,
        out_shape=[plane] * 6,
    )(coarse_input.T, coarse_pred.T, fine_input.T, fine_pred.T)

    idxF = jnp.transpose(laplace_idx_list, (2, 0, 1)).reshape(-1)

    mesh = plsc.VectorSubcoreMesh(core_axis_name="c", subcore_axis_name="s")
    part = pl.kernel(
        _sc_body,
        mesh=mesh,
        compiler_params=pltpu.CompilerParams(needs_layout_passes=False),
        out_type=jax.ShapeDtypeStruct((NW * 16,), jnp.float32),
        scratch_types=[
            pltpu.VMEM((N,), jnp.float32),
            pltpu.VMEM((N,), jnp.float32),
            pltpu.VMEM(((KNB + 1) * BPW,), jnp.int32),
            pltpu.VMEM((BPW,), jnp.float32),
            pltpu.VMEM((16,), jnp.float32),
            pltpu.SemaphoreType.DMA,
            pltpu.SemaphoreType.DMA,
        ],
    )(*d_planes, idxF)
    return jnp.sum(part) * jnp.float32(0.5 / N)
